# Initial kernel scaffold; baseline (speedup 1.0000x reference)
import jax
import jax.numpy as jnp
from jax.experimental import pallas as pl


def kernel(adj, features_u, features_v, W0, W1, W2):
    raise NotImplementedError("write your pallas kernel here")


# speedup vs baseline: 1.2073x; 1.2073x over previous
import functools

import jax
import jax.numpy as jnp
from jax.experimental import pallas as pl
from jax.experimental.pallas import tpu as pltpu

_BU = 400


def _d_kernel(adj_ref, du_ref, dv_ref):
    x = adj_ref[...]
    du_ref[...] = jax.lax.rsqrt(jnp.sum(x, axis=1, keepdims=True) + 1.0)

    @pl.when(pl.program_id(0) == 0)
    def _():
        dv_ref[...] = jnp.zeros_like(dv_ref)

    dv_ref[...] += jnp.sum(x, axis=0, keepdims=True)

    @pl.when(pl.program_id(0) == pl.num_programs(0) - 1)
    def _():
        dv_ref[...] = jax.lax.rsqrt(dv_ref[...] + 1.0)


def _layer_kernel(adj_ref, hu_ref, hv_ref, du_ref, dv_ref, w_ref,
                  huo_ref, hvo_ref, hvsw_ref):
    @pl.when(pl.program_id(0) == 0)
    def _():
        hvsw_ref[...] = jax.lax.dot_general(
            dv_ref[...] * hv_ref[...], w_ref[...],
            (((1,), (0,)), ((), ())), preferred_element_type=jnp.float32)
        hvo_ref[...] = jnp.zeros_like(hvo_ref)

    x = adj_ref[...]
    mu = jax.lax.dot_general(x, hvsw_ref[...], (((1,), (0,)), ((), ())),
                             preferred_element_type=jnp.float32)
    huo_ref[...] = jnp.maximum(du_ref[...] * mu, 0.0)

    husw = jax.lax.dot_general(du_ref[...] * hu_ref[...], w_ref[...],
                               (((1,), (0,)), ((), ())),
                               preferred_element_type=jnp.float32)
    hvo_ref[...] += jax.lax.dot_general(x, husw, (((0,), (0,)), ((), ())),
                                        preferred_element_type=jnp.float32)

    @pl.when(pl.program_id(0) == pl.num_programs(0) - 1)
    def _():
        hvo_ref[...] = jnp.maximum(dv_ref[...] * hvo_ref[...], 0.0)


def _out_kernel(adj_ref, hv_ref, du_ref, dv_ref, w_ref, out_ref, hvsw_ref,
                *, n_cls):
    @pl.when(pl.program_id(0) == 0)
    def _():
        hvsw_ref[...] = jax.lax.dot_general(
            dv_ref[...] * hv_ref[...], w_ref[...],
            (((1,), (0,)), ((), ())), preferred_element_type=jnp.float32)

    x = adj_ref[...]
    logits = du_ref[...] * jax.lax.dot_general(
        x, hvsw_ref[...], (((1,), (0,)), ((), ())),
        preferred_element_type=jnp.float32)
    col = jax.lax.broadcasted_iota(jnp.int32, logits.shape, 1)
    mask = col < n_cls
    masked = jnp.where(mask, logits, -1e30)
    m = jnp.max(masked, axis=1, keepdims=True)
    e = jnp.where(mask, jnp.exp(logits - m), 0.0)
    s = jnp.sum(e, axis=1, keepdims=True)
    out_ref[...] = (logits - m - jnp.log(s))[:, :n_cls]


def kernel(adj, features_u, features_v, W0, W1, W2):
    n_u, n_v = adj.shape
    d_h = W0.shape[1]
    n_cls = W2.shape[1]
    bu = _BU if n_u % _BU == 0 else n_u
    grid = (n_u // bu,)

    du, dv_row = pl.pallas_call(
        _d_kernel,
        grid=grid,
        in_specs=[pl.BlockSpec((bu, n_v), lambda i: (i, 0))],
        out_specs=[pl.BlockSpec((bu, 1), lambda i: (i, 0)),
                   pl.BlockSpec((1, n_v), lambda i: (0, 0))],
        out_shape=[jax.ShapeDtypeStruct((n_u, 1), jnp.float32),
                   jax.ShapeDtypeStruct((1, n_v), jnp.float32)],
    )(adj)
    dv = dv_row.reshape(n_v, 1)

    def layer(hu, hv, W, d_o):
        d_i = hu.shape[1]
        return pl.pallas_call(
            _layer_kernel,
            grid=grid,
            in_specs=[
                pl.BlockSpec((bu, n_v), lambda i: (i, 0)),
                pl.BlockSpec((bu, d_i), lambda i: (i, 0)),
                pl.BlockSpec((n_v, d_i), lambda i: (0, 0)),
                pl.BlockSpec((bu, 1), lambda i: (i, 0)),
                pl.BlockSpec((n_v, 1), lambda i: (0, 0)),
                pl.BlockSpec((d_i, d_o), lambda i: (0, 0)),
            ],
            out_specs=[pl.BlockSpec((bu, d_o), lambda i: (i, 0)),
                       pl.BlockSpec((n_v, d_o), lambda i: (0, 0))],
            out_shape=[jax.ShapeDtypeStruct((n_u, d_o), jnp.float32),
                       jax.ShapeDtypeStruct((n_v, d_o), jnp.float32)],
            scratch_shapes=[pltpu.VMEM((n_v, d_o), jnp.float32)],
        )(adj, hu, hv, du, dv, W)

    hu1, hv1 = layer(features_u, features_v, W0, d_h)
    hu2, hv2 = layer(hu1, hv1, W1, d_h)

    w2p = jnp.zeros((d_h, d_h), jnp.float32).at[:, :n_cls].set(W2)

    logp = pl.pallas_call(
        functools.partial(_out_kernel, n_cls=n_cls),
        grid=grid,
        in_specs=[
            pl.BlockSpec((bu, n_v), lambda i: (i, 0)),
            pl.BlockSpec((n_v, d_h), lambda i: (0, 0)),
            pl.BlockSpec((bu, 1), lambda i: (i, 0)),
            pl.BlockSpec((n_v, 1), lambda i: (0, 0)),
            pl.BlockSpec((d_h, d_h), lambda i: (0, 0)),
        ],
        out_specs=pl.BlockSpec((bu, n_cls), lambda i: (i, 0)),
        out_shape=jax.ShapeDtypeStruct((n_u, n_cls), jnp.float32),
        scratch_shapes=[pltpu.VMEM((n_v, d_h), jnp.float32)],
    )(adj, hv2, du, dv, w2p)

    return logp, hu2


# speedup vs baseline: 1.2588x; 1.0426x over previous
import functools

import jax
import jax.numpy as jnp
from jax.experimental import pallas as pl
from jax.experimental.pallas import tpu as pltpu

_BU = 400


def _dot(a, b):
    return jax.lax.dot_general(a, b, (((1,), (0,)), ((), ())),
                               preferred_element_type=jnp.float32)


def _dot_t(a, b):
    return jax.lax.dot_general(a, b, (((0,), (0,)), ((), ())),
                               preferred_element_type=jnp.float32)


def _pass_a(adj_ref, hu0_ref, w0_ref, abf_ref, du_ref, dv_ref, hv1_ref,
            acc_ref):
    xf = adj_ref[...]
    xb = xf.astype(jnp.bfloat16)
    abf_ref[...] = xb
    du = jax.lax.rsqrt(jnp.sum(xf, axis=1, keepdims=True) + 1.0)
    du_ref[...] = du

    t0 = _dot(du * hu0_ref[...], w0_ref[...])
    bu, d = t0.shape
    ones_col = jnp.ones((bu, 1), jnp.float32)
    pad = jnp.zeros((bu, d - 1), jnp.float32)
    t_aug = jnp.concatenate([t0, ones_col, pad], axis=1).astype(jnp.bfloat16)

    @pl.when(pl.program_id(0) == 0)
    def _():
        acc_ref[...] = jnp.zeros_like(acc_ref)

    acc_ref[...] += _dot_t(xb, t_aug)

    @pl.when(pl.program_id(0) == pl.num_programs(0) - 1)
    def _():
        acc = acc_ref[...]
        dv = jax.lax.rsqrt(acc[:, d:d + 1] + 1.0)
        dv_ref[...] = dv
        hv1_ref[...] = jnp.maximum(dv * acc[:, :d], 0.0)


def _pass_b(abf_ref, hv0_ref, hv1_ref, du_ref, dv_ref, w0_ref, w1_ref,
            hu2_ref, hv2_ref, sw_ref, acc_ref):
    d = w0_ref.shape[1]

    @pl.when(pl.program_id(0) == 0)
    def _():
        dv = dv_ref[...]
        s0 = _dot(dv * hv0_ref[...], w0_ref[...])
        s1 = _dot(dv * hv1_ref[...], w1_ref[...])
        sw_ref[...] = jnp.concatenate([s0, s1], axis=1).astype(jnp.bfloat16)
        acc_ref[...] = jnp.zeros_like(acc_ref)

    x = abf_ref[...]
    z = _dot(x, sw_ref[...])
    du = du_ref[...]
    hu1 = jnp.maximum(du * z[:, :d], 0.0)
    hu2_ref[...] = jnp.maximum(du * z[:, d:], 0.0)

    t1 = _dot(du * hu1, w1_ref[...]).astype(jnp.bfloat16)
    acc_ref[...] += _dot_t(x, t1)

    @pl.when(pl.program_id(0) == pl.num_programs(0) - 1)
    def _():
        hv2_ref[...] = jnp.maximum(dv_ref[...] * acc_ref[...], 0.0)


def _pass_c(abf_ref, hv2_ref, du_ref, dv_ref, w2_ref, out_ref, sw_ref,
            *, n_cls):
    @pl.when(pl.program_id(0) == 0)
    def _():
        sw_ref[...] = _dot(dv_ref[...] * hv2_ref[...],
                           w2_ref[...]).astype(jnp.bfloat16)

    logits = du_ref[...] * _dot(abf_ref[...], sw_ref[...])
    col = jax.lax.broadcasted_iota(jnp.int32, logits.shape, 1)
    mask = col < n_cls
    masked = jnp.where(mask, logits, -1e30)
    m = jnp.max(masked, axis=1, keepdims=True)
    e = jnp.where(mask, jnp.exp(logits - m), 0.0)
    s = jnp.sum(e, axis=1, keepdims=True)
    out_ref[...] = (logits - m - jnp.log(s))[:, :n_cls]


def kernel(adj, features_u, features_v, W0, W1, W2):
    n_u, n_v = adj.shape
    d_h = W0.shape[1]
    n_cls = W2.shape[1]
    bu = _BU if n_u % _BU == 0 else n_u
    grid = (n_u // bu,)
    f32 = jnp.float32

    abf, du, dv, hv1 = pl.pallas_call(
        _pass_a,
        grid=grid,
        in_specs=[
            pl.BlockSpec((bu, n_v), lambda i: (i, 0)),
            pl.BlockSpec((bu, d_h), lambda i: (i, 0)),
            pl.BlockSpec((d_h, d_h), lambda i: (0, 0)),
        ],
        out_specs=[
            pl.BlockSpec((bu, n_v), lambda i: (i, 0)),
            pl.BlockSpec((bu, 1), lambda i: (i, 0)),
            pl.BlockSpec((n_v, 1), lambda i: (0, 0)),
            pl.BlockSpec((n_v, d_h), lambda i: (0, 0)),
        ],
        out_shape=[
            jax.ShapeDtypeStruct((n_u, n_v), jnp.bfloat16),
            jax.ShapeDtypeStruct((n_u, 1), f32),
            jax.ShapeDtypeStruct((n_v, 1), f32),
            jax.ShapeDtypeStruct((n_v, d_h), f32),
        ],
        scratch_shapes=[pltpu.VMEM((n_v, 2 * d_h), f32)],
    )(adj, features_u, W0)

    hu2, hv2 = pl.pallas_call(
        _pass_b,
        grid=grid,
        in_specs=[
            pl.BlockSpec((bu, n_v), lambda i: (i, 0)),
            pl.BlockSpec((n_v, d_h), lambda i: (0, 0)),
            pl.BlockSpec((n_v, d_h), lambda i: (0, 0)),
            pl.BlockSpec((bu, 1), lambda i: (i, 0)),
            pl.BlockSpec((n_v, 1), lambda i: (0, 0)),
            pl.BlockSpec((d_h, d_h), lambda i: (0, 0)),
            pl.BlockSpec((d_h, d_h), lambda i: (0, 0)),
        ],
        out_specs=[
            pl.BlockSpec((bu, d_h), lambda i: (i, 0)),
            pl.BlockSpec((n_v, d_h), lambda i: (0, 0)),
        ],
        out_shape=[
            jax.ShapeDtypeStruct((n_u, d_h), f32),
            jax.ShapeDtypeStruct((n_v, d_h), f32),
        ],
        scratch_shapes=[pltpu.VMEM((n_v, 2 * d_h), jnp.bfloat16),
                        pltpu.VMEM((n_v, d_h), f32)],
    )(abf, features_v, hv1, du, dv, W0, W1)

    w2p = jnp.zeros((d_h, d_h), f32).at[:, :n_cls].set(W2)

    logp = pl.pallas_call(
        functools.partial(_pass_c, n_cls=n_cls),
        grid=grid,
        in_specs=[
            pl.BlockSpec((bu, n_v), lambda i: (i, 0)),
            pl.BlockSpec((n_v, d_h), lambda i: (0, 0)),
            pl.BlockSpec((bu, 1), lambda i: (i, 0)),
            pl.BlockSpec((n_v, 1), lambda i: (0, 0)),
            pl.BlockSpec((d_h, d_h), lambda i: (0, 0)),
        ],
        out_specs=pl.BlockSpec((bu, n_cls), lambda i: (i, 0)),
        out_shape=jax.ShapeDtypeStruct((n_u, n_cls), f32),
        scratch_shapes=[pltpu.VMEM((n_v, d_h), jnp.bfloat16)],
    )(abf, hv2, du, dv, w2p)

    return logp, hu2


# speedup vs baseline: 1.3941x; 1.1075x over previous
import functools

import jax
import jax.numpy as jnp
from jax.experimental import pallas as pl
from jax.experimental.pallas import tpu as pltpu

_BU = 400
_BV = 512


def _dot(a, b):
    return jax.lax.dot_general(a, b, (((1,), (0,)), ((), ())),
                               preferred_element_type=jnp.float32)


def _dot_t(a, b):
    return jax.lax.dot_general(a, b, (((0,), (0,)), ((), ())),
                               preferred_element_type=jnp.float32)


def _pass_a(adj_ref, hu0_ref, w0_ref, abf_ref, du_ref, dv_ref, hv1_ref,
            acc_ref):
    xf = adj_ref[...]
    xb = xf.astype(jnp.bfloat16)
    abf_ref[...] = xb
    du = jax.lax.rsqrt(jnp.sum(xf, axis=1, keepdims=True) + 1.0)
    du_ref[...] = du

    t0 = _dot(du * hu0_ref[...], w0_ref[...])
    bu, d = t0.shape
    ones_col = jnp.ones((bu, 1), jnp.float32)
    pad = jnp.zeros((bu, d - 1), jnp.float32)
    t_aug = jnp.concatenate([t0, ones_col, pad], axis=1).astype(jnp.bfloat16)

    @pl.when(pl.program_id(0) == 0)
    def _():
        acc_ref[...] = jnp.zeros_like(acc_ref)

    acc_ref[...] += _dot_t(xb, t_aug)

    @pl.when(pl.program_id(0) == pl.num_programs(0) - 1)
    def _():
        acc = acc_ref[...]
        dv = jax.lax.rsqrt(acc[:, d:d + 1] + 1.0)
        dv_ref[...] = dv
        hv1_ref[...] = jnp.maximum(dv * acc[:, :d], 0.0)


def _pass_b1(abf_ref, hv0_ref, hv1_ref, du_ref, dv_ref, w0_ref, w1_ref,
             hu2_ref, t1_ref, sw_ref):
    d = w0_ref.shape[1]

    @pl.when(pl.program_id(0) == 0)
    def _():
        dv = dv_ref[...]
        s0 = _dot(dv * hv0_ref[...], w0_ref[...])
        s1 = _dot(dv * hv1_ref[...], w1_ref[...])
        sw_ref[...] = jnp.concatenate([s0, s1], axis=1).astype(jnp.bfloat16)

    x = abf_ref[...]
    z = _dot(x, sw_ref[...])
    du = du_ref[...]
    hu1 = jnp.maximum(du * z[:, :d], 0.0)
    hu2_ref[...] = jnp.maximum(du * z[:, d:], 0.0)
    t1_ref[...] = _dot(du * hu1, w1_ref[...]).astype(jnp.bfloat16)


def _pass_b2(abf_ref, t1_ref, dv_ref, hv2_ref):
    agg = _dot_t(abf_ref[...], t1_ref[...])
    hv2_ref[...] = jnp.maximum(dv_ref[...] * agg, 0.0)


def _pass_c(abf_ref, hv2_ref, du_ref, dv_ref, w2_ref, out_ref, sw_ref,
            *, n_cls):
    @pl.when(pl.program_id(0) == 0)
    def _():
        sw_ref[...] = _dot(dv_ref[...] * hv2_ref[...],
                           w2_ref[...]).astype(jnp.bfloat16)

    logits = du_ref[...] * _dot(abf_ref[...], sw_ref[...])
    col = jax.lax.broadcasted_iota(jnp.int32, logits.shape, 1)
    mask = col < n_cls
    masked = jnp.where(mask, logits, -1e30)
    m = jnp.max(masked, axis=1, keepdims=True)
    e = jnp.where(mask, jnp.exp(logits - m), 0.0)
    s = jnp.sum(e, axis=1, keepdims=True)
    out_ref[...] = (logits - m - jnp.log(s))[:, :n_cls]


def kernel(adj, features_u, features_v, W0, W1, W2):
    n_u, n_v = adj.shape
    d_h = W0.shape[1]
    n_cls = W2.shape[1]
    bu = _BU if n_u % _BU == 0 else n_u
    bv = _BV if n_v % _BV == 0 else n_v
    grid = (n_u // bu,)
    f32 = jnp.float32

    abf, du, dv, hv1 = pl.pallas_call(
        _pass_a,
        grid=grid,
        in_specs=[
            pl.BlockSpec((bu, n_v), lambda i: (i, 0)),
            pl.BlockSpec((bu, d_h), lambda i: (i, 0)),
            pl.BlockSpec((d_h, d_h), lambda i: (0, 0)),
        ],
        out_specs=[
            pl.BlockSpec((bu, n_v), lambda i: (i, 0)),
            pl.BlockSpec((bu, 1), lambda i: (i, 0)),
            pl.BlockSpec((n_v, 1), lambda i: (0, 0)),
            pl.BlockSpec((n_v, d_h), lambda i: (0, 0)),
        ],
        out_shape=[
            jax.ShapeDtypeStruct((n_u, n_v), jnp.bfloat16),
            jax.ShapeDtypeStruct((n_u, 1), f32),
            jax.ShapeDtypeStruct((n_v, 1), f32),
            jax.ShapeDtypeStruct((n_v, d_h), f32),
        ],
        scratch_shapes=[pltpu.VMEM((n_v, 2 * d_h), f32)],
    )(adj, features_u, W0)

    hu2, t1 = pl.pallas_call(
        _pass_b1,
        grid=grid,
        in_specs=[
            pl.BlockSpec((bu, n_v), lambda i: (i, 0)),
            pl.BlockSpec((n_v, d_h), lambda i: (0, 0)),
            pl.BlockSpec((n_v, d_h), lambda i: (0, 0)),
            pl.BlockSpec((bu, 1), lambda i: (i, 0)),
            pl.BlockSpec((n_v, 1), lambda i: (0, 0)),
            pl.BlockSpec((d_h, d_h), lambda i: (0, 0)),
            pl.BlockSpec((d_h, d_h), lambda i: (0, 0)),
        ],
        out_specs=[
            pl.BlockSpec((bu, d_h), lambda i: (i, 0)),
            pl.BlockSpec((bu, d_h), lambda i: (i, 0)),
        ],
        out_shape=[
            jax.ShapeDtypeStruct((n_u, d_h), f32),
            jax.ShapeDtypeStruct((n_u, d_h), jnp.bfloat16),
        ],
        scratch_shapes=[pltpu.VMEM((n_v, 2 * d_h), jnp.bfloat16)],
    )(abf, features_v, hv1, du, dv, W0, W1)

    hv2 = pl.pallas_call(
        _pass_b2,
        grid=(n_v // bv,),
        in_specs=[
            pl.BlockSpec((n_u, bv), lambda j: (0, j)),
            pl.BlockSpec((n_u, d_h), lambda j: (0, 0)),
            pl.BlockSpec((bv, 1), lambda j: (j, 0)),
        ],
        out_specs=pl.BlockSpec((bv, d_h), lambda j: (j, 0)),
        out_shape=jax.ShapeDtypeStruct((n_v, d_h), f32),
    )(abf, t1, dv)

    w2p = jnp.zeros((d_h, d_h), f32).at[:, :n_cls].set(W2)

    logp = pl.pallas_call(
        functools.partial(_pass_c, n_cls=n_cls),
        grid=grid,
        in_specs=[
            pl.BlockSpec((bu, n_v), lambda i: (i, 0)),
            pl.BlockSpec((n_v, d_h), lambda i: (0, 0)),
            pl.BlockSpec((bu, 1), lambda i: (i, 0)),
            pl.BlockSpec((n_v, 1), lambda i: (0, 0)),
            pl.BlockSpec((d_h, d_h), lambda i: (0, 0)),
        ],
        out_specs=pl.BlockSpec((bu, n_cls), lambda i: (i, 0)),
        out_shape=jax.ShapeDtypeStruct((n_u, n_cls), f32),
        scratch_shapes=[pltpu.VMEM((n_v, d_h), jnp.bfloat16)],
    )(abf, hv2, du, dv, w2p)

    return logp, hu2
